# baseline (device time: 14441 ns/iter reference)
import jax
import jax.numpy as jnp
from jax import lax
from jax.experimental import pallas as pl
from jax.experimental.pallas import tpu as pltpu

N_DEV = 16
N_CHUNK = 4


def kernel(x):
    m_per, n = x.shape
    m_c = m_per // N_CHUNK

    def body(x_hbm, out_ref, xbuf, comm_ref, copy_sems, send_sems, recv_sems):
        my_pos = lax.axis_index("i")

        barrier_sem = pltpu.get_barrier_semaphore()
        for d in range(1, N_DEV):
            pl.semaphore_signal(
                barrier_sem,
                inc=1,
                device_id=((my_pos + d) % N_DEV,),
                device_id_type=pl.DeviceIdType.MESH,
            )

        copies = []
        for c in range(N_CHUNK):
            cp = pltpu.make_async_copy(
                x_hbm.at[pl.ds(c * m_c, m_c), :],
                xbuf.at[c],
                copy_sems.at[c],
            )
            cp.start()
            copies.append(cp)

        copies[0].wait()
        run = jnp.max(xbuf[0], axis=0, keepdims=True)
        for c in range(1, N_CHUNK):
            copies[c].wait()
            run = jnp.maximum(run, jnp.max(xbuf[c], axis=0, keepdims=True))
        comm_ref[0, 0:1, :] = run

        pl.semaphore_wait(barrier_sem, N_DEV - 1)

        def start_sends(row):
            rdmas = []
            for d in range(1, N_DEV):
                rdma = pltpu.make_async_remote_copy(
                    src_ref=comm_ref.at[0, pl.ds(row, 1), :],
                    dst_ref=comm_ref.at[d, pl.ds(row, 1), :],
                    send_sem=send_sems.at[row, d],
                    recv_sem=recv_sems.at[row, d],
                    device_id=((my_pos + d) % N_DEV,),
                    device_id_type=pl.DeviceIdType.MESH,
                )
                rdma.start()
                rdmas.append(rdma)
            return rdmas

        rdmas_val = start_sends(0)

        val = comm_ref[0, 0:1, :]
        loc = jnp.full((1, n), 2**30, dtype=jnp.int32)
        for c in range(N_CHUNK):
            iota = lax.broadcasted_iota(jnp.int32, (m_c, n), 0) + c * m_c
            masked = jnp.where(xbuf[c] == val, iota, jnp.int32(2**30))
            loc = jnp.minimum(loc, jnp.min(masked, axis=0, keepdims=True))
        comm_ref[0, 1:2, :] = (loc + my_pos * m_per).astype(jnp.float32)

        rdmas_idx = start_sends(1)
        for rdma in rdmas_val + rdmas_idx:
            rdma.wait()

        vals = comm_ref[:, 0, :]
        idxs = comm_ref[:, 1, :]
        best = jnp.max(vals, axis=0, keepdims=True)
        cand = jnp.where(vals == best, idxs, jnp.float32(jnp.inf))
        out_ref[0:1, :] = best
        out_ref[1:2, :] = jnp.min(cand, axis=0, keepdims=True)

    return pl.pallas_call(
        body,
        out_shape=jax.ShapeDtypeStruct((2, n), jnp.float32),
        in_specs=[pl.BlockSpec(memory_space=pl.ANY)],
        out_specs=pl.BlockSpec(memory_space=pltpu.VMEM),
        scratch_shapes=[
            pltpu.VMEM((N_CHUNK, m_c, n), jnp.float32),
            pltpu.VMEM((N_DEV, 2, n), jnp.float32),
            pltpu.SemaphoreType.DMA((N_CHUNK,)),
            pltpu.SemaphoreType.DMA((2, N_DEV)),
            pltpu.SemaphoreType.DMA((2, N_DEV)),
        ],
        compiler_params=pltpu.CompilerParams(collective_id=0),
    )(x)


# device time: 14396 ns/iter; 1.0031x vs baseline; 1.0031x over previous
import jax
import jax.numpy as jnp
from jax import lax
from jax.experimental import pallas as pl
from jax.experimental.pallas import tpu as pltpu

N_DEV = 16
N_CHUNK = 4


def kernel(x):
    m_per, n = x.shape
    m_c = m_per // N_CHUNK

    def body(x_hbm, out_ref, xbuf, comm_ref, copy_sems, send_sems, recv_sems):
        my_pos = lax.axis_index("i")

        barrier_sem = pltpu.get_barrier_semaphore()
        for d in range(1, N_DEV):
            pl.semaphore_signal(
                barrier_sem,
                inc=1,
                device_id=((my_pos + d) % N_DEV,),
                device_id_type=pl.DeviceIdType.MESH,
            )

        copies = []
        for c in range(N_CHUNK):
            cp = pltpu.make_async_copy(
                x_hbm.at[pl.ds(c * m_c, m_c), :],
                xbuf.at[c],
                copy_sems.at[c],
            )
            cp.start()
            copies.append(cp)

        copies[0].wait()
        run = jnp.max(xbuf[0], axis=0, keepdims=True)
        for c in range(1, N_CHUNK):
            copies[c].wait()
            run = jnp.maximum(run, jnp.max(xbuf[c], axis=0, keepdims=True))
        comm_ref[0, 0:1, :] = run

        val = comm_ref[0, 0:1, :]
        loc = jnp.full((1, n), 2**30, dtype=jnp.int32)
        for c in range(N_CHUNK):
            iota = lax.broadcasted_iota(jnp.int32, (m_c, n), 0) + c * m_c
            masked = jnp.where(xbuf[c] == val, iota, jnp.int32(2**30))
            loc = jnp.minimum(loc, jnp.min(masked, axis=0, keepdims=True))
        comm_ref[0, 1:2, :] = (loc + my_pos * m_per).astype(jnp.float32)

        pl.semaphore_wait(barrier_sem, N_DEV - 1)
        rdmas = []
        for d in range(1, N_DEV):
            rdma = pltpu.make_async_remote_copy(
                src_ref=comm_ref.at[0],
                dst_ref=comm_ref.at[d],
                send_sem=send_sems.at[0, d],
                recv_sem=recv_sems.at[0, d],
                device_id=((my_pos + d) % N_DEV,),
                device_id_type=pl.DeviceIdType.MESH,
            )
            rdma.start()
            rdmas.append(rdma)
        for rdma in rdmas:
            rdma.wait()

        vals = comm_ref[:, 0, :]
        idxs = comm_ref[:, 1, :]
        best = jnp.max(vals, axis=0, keepdims=True)
        cand = jnp.where(vals == best, idxs, jnp.float32(jnp.inf))
        out_ref[0:1, :] = best
        out_ref[1:2, :] = jnp.min(cand, axis=0, keepdims=True)

    return pl.pallas_call(
        body,
        out_shape=jax.ShapeDtypeStruct((2, n), jnp.float32),
        in_specs=[pl.BlockSpec(memory_space=pl.ANY)],
        out_specs=pl.BlockSpec(memory_space=pltpu.VMEM),
        scratch_shapes=[
            pltpu.VMEM((N_CHUNK, m_c, n), jnp.float32),
            pltpu.VMEM((N_DEV, 2, n), jnp.float32),
            pltpu.SemaphoreType.DMA((N_CHUNK,)),
            pltpu.SemaphoreType.DMA((2, N_DEV)),
            pltpu.SemaphoreType.DMA((2, N_DEV)),
        ],
        compiler_params=pltpu.CompilerParams(collective_id=0),
    )(x)


# device time: 13994 ns/iter; 1.0319x vs baseline; 1.0287x over previous
import jax
import jax.numpy as jnp
from jax import lax
from jax.experimental import pallas as pl
from jax.experimental.pallas import tpu as pltpu

N_DEV = 16


def kernel(x):
    m_per, n = x.shape

    def body(x_ref, out_ref, comm_ref, send_sems, recv_sems):
        my_pos = lax.axis_index("i")

        barrier_sem = pltpu.get_barrier_semaphore()
        for d in range(1, N_DEV):
            pl.semaphore_signal(
                barrier_sem,
                inc=1,
                device_id=((my_pos + d) % N_DEV,),
                device_id_type=pl.DeviceIdType.MESH,
            )

        xv = x_ref[:, :]
        val = jnp.max(xv, axis=0, keepdims=True)
        loc = jnp.argmax(xv, axis=0).reshape(1, n)
        comm_ref[0, 0:1, :] = val
        comm_ref[0, 1:2, :] = (loc + my_pos * m_per).astype(jnp.float32)

        pl.semaphore_wait(barrier_sem, N_DEV - 1)

        rdmas = []
        for d in range(1, N_DEV):
            rdma = pltpu.make_async_remote_copy(
                src_ref=comm_ref.at[0],
                dst_ref=comm_ref.at[d],
                send_sem=send_sems.at[d],
                recv_sem=recv_sems.at[d],
                device_id=((my_pos + d) % N_DEV,),
                device_id_type=pl.DeviceIdType.MESH,
            )
            rdma.start()
            rdmas.append(rdma)
        for rdma in rdmas:
            rdma.wait()

        vals = comm_ref[:, 0, :]
        idxs = comm_ref[:, 1, :]
        best = jnp.max(vals, axis=0, keepdims=True)
        cand = jnp.where(vals == best, idxs, jnp.float32(jnp.inf))
        out_ref[0:1, :] = best
        out_ref[1:2, :] = jnp.min(cand, axis=0, keepdims=True)

    return pl.pallas_call(
        body,
        out_shape=jax.ShapeDtypeStruct((2, n), jnp.float32),
        in_specs=[pl.BlockSpec(memory_space=pltpu.VMEM)],
        out_specs=pl.BlockSpec(memory_space=pltpu.VMEM),
        scratch_shapes=[
            pltpu.VMEM((N_DEV, 2, n), jnp.float32),
            pltpu.SemaphoreType.DMA((N_DEV,)),
            pltpu.SemaphoreType.DMA((N_DEV,)),
        ],
        compiler_params=pltpu.CompilerParams(collective_id=0),
    )(x)
